# baseline (speedup 1.0000x reference)
import functools
import jax
import jax.numpy as jnp
from jax import lax
from jax.experimental import pallas as pl
from jax.experimental.pallas import tpu as pltpu
from jax.experimental.pallas import tpu_sc as plsc

BATCH = 16384
HIST = 50
EMBED_DIM = 64

NUM_WORKERS = 32
BPW = BATCH // NUM_WORKERS
CHUNK = 128
SUBW = BPW // CHUNK
NUNITS = HIST * SUBW
KSUB = CHUNK // 128
PITCH = CHUNK + 8
PITCH_G = 132


def _sc_gather(xt, table2):
    mesh = plsc.VectorSubcoreMesh(core_axis_name="c", subcore_axis_name="s")

    @functools.partial(
        pl.kernel,
        mesh=mesh,
        out_type=jax.ShapeDtypeStruct((HIST, EMBED_DIM, BATCH), jnp.float32),
        scratch_types=[
            pltpu.VMEM((2, CHUNK), jnp.int32),
            pltpu.VMEM((2, KSUB, 128), jnp.int32),
            pltpu.VMEM((2, CHUNK + 16), jnp.int32),
            pltpu.VMEM((2, CHUNK, 128), jnp.float32),
            pltpu.VMEM((2, CHUNK, EMBED_DIM), jnp.float32),
            pltpu.SemaphoreType.DMA((2,)),
            pltpu.SemaphoreType.DMA((2,)),
        ],
        compiler_params=pltpu.CompilerParams(needs_layout_passes=False),
    )
    def k(x_hbm, t_hbm, out_hbm, idx_v, idx2_v, par_v, big_v, rows_v,
          gsem, wsem):
        wid = lax.axis_index("s") * 2 + lax.axis_index("c")
        lanes = lax.iota(jnp.int32, 16)

        def unit_hb(u):
            return u // SUBW, wid * BPW + (u % SUBW) * CHUNK

        def load_and_prep(u, b):
            h, boff = unit_hb(u)
            pltpu.sync_copy(x_hbm.at[h, pl.ds(boff, CHUNK)], idx_v.at[b])
            for i in range(CHUNK // 16):
                v = idx_v[b, pl.ds(i * 16, 16)]
                idx2_v[b, i // 8, pl.ds((i % 8) * 16, 16)] = (
                    lax.shift_right_logical(v, 1))
                par_v[b, pl.ds(i * 16, 16)] = (v & 1) * 64

        def fire_gathers(b):
            for j in range(KSUB):
                pltpu.async_copy(
                    t_hbm.at[idx2_v.at[b, j]],
                    big_v.at[b, pl.ds(j * 128, 128)],
                    gsem.at[b])

        def wait_gathers(b):
            for j in range(KSUB):
                pltpu.make_async_copy(
                    t_hbm.at[idx2_v.at[b, j]],
                    big_v.at[b, pl.ds(j * 128, 128)],
                    gsem.at[b]).wait()

        def transpose(b):
            @plsc.parallel_loop(0, CHUNK, unroll=4)
            def body(bb):
                pv = par_v[b, pl.ds(bb, 16)][0]
                for c0 in range(EMBED_DIM // 16):
                    v = plsc.load_gather(
                        big_v.at[b], [jnp.full((16,), bb, jnp.int32),
                                      pv + c0 * 16 + lanes])
                    rows_v[b, bb, pl.ds(c0 * 16, 16)] = v

        def start_write(u, b):
            h, boff = unit_hb(u)
            for c in range(EMBED_DIM):
                pltpu.async_copy(
                    rows_v.at[b, slice(None), c],
                    out_hbm.at[h, c, pl.ds(boff, CHUNK)],
                    wsem.at[b])

        def wait_write(u, b):
            h, boff = unit_hb(u)
            for c in range(EMBED_DIM):
                pltpu.make_async_copy(
                    rows_v.at[b, slice(None), c],
                    out_hbm.at[h, c, pl.ds(boff, CHUNK)],
                    wsem.at[b]).wait()

        load_and_prep(0, 0)
        fire_gathers(0)
        load_and_prep(1, 1)
        fire_gathers(1)
        for b in range(2):
            wait_gathers(b)
            transpose(b)
            start_write(b, b)
            load_and_prep(b + 2, b)
            fire_gathers(b)

        def step(t, carry):
            for b in range(2):
                u = 2 * t + b
                wait_gathers(b)
                wait_write(u - 2, b)
                transpose(b)
                start_write(u, b)
                load_and_prep(u + 2, b)
                fire_gathers(b)
            return carry

        lax.fori_loop(1, NUNITS // 2 - 1, step, 0)

        for b in range(2):
            u = NUNITS - 2 + b
            wait_gathers(b)
            wait_write(u - 2, b)
            transpose(b)
            start_write(u, b)
        for b in range(2):
            wait_write(NUNITS - 2 + b, b)

    return k(xt, table2)


def kernel(x, table):
    xt = jnp.transpose(x.astype(jnp.int32))
    table2 = jnp.reshape(table, (table.shape[0] // 2, 128))
    out = _sc_gather(xt, table2)
    return jnp.transpose(out, (2, 0, 1))


# speedup vs baseline: 94.9756x; 94.9756x over previous
import functools
import jax
import jax.numpy as jnp
from jax import lax
from jax.experimental import pallas as pl
from jax.experimental.pallas import tpu as pltpu
from jax.experimental.pallas import tpu_sc as plsc

BATCH = 16384
HIST = 50
EMBED_DIM = 64
TOTAL = BATCH * HIST

NUM_WORKERS = 32
ROWS_PER_WORKER = TOTAL // NUM_WORKERS
CHUNK = 512
KSUB = CHUNK // 128
NCHUNKS = ROWS_PER_WORKER // CHUNK


def _sc_gather(x2d, table):
    mesh = plsc.VectorSubcoreMesh(core_axis_name="c", subcore_axis_name="s")

    @functools.partial(
        pl.kernel,
        mesh=mesh,
        out_type=jax.ShapeDtypeStruct((TOTAL, EMBED_DIM), jnp.float32),
        scratch_types=[
            pltpu.VMEM((2, KSUB, 128), jnp.int32),
            pltpu.VMEM((2, CHUNK, EMBED_DIM), jnp.float32),
            pltpu.SemaphoreType.DMA((2,)),
            pltpu.SemaphoreType.DMA((2,)),
        ],
        compiler_params=pltpu.CompilerParams(use_tc_tiling_on_sc=False),
    )
    def k(x_hbm, table_hbm, out_hbm, idx_v, rows_v, gsem, wsem):
        wid = lax.axis_index("s") * 2 + lax.axis_index("c")
        row0 = wid * (ROWS_PER_WORKER // 128)

        def fire_gathers(ch, b):
            for j in range(KSUB):
                pltpu.async_copy(
                    table_hbm.at[idx_v.at[b, j]],
                    rows_v.at[b, pl.ds(j * 128, 128)],
                    gsem.at[b])

        def wait_gathers(ch, b):
            for j in range(KSUB):
                pltpu.make_async_copy(
                    table_hbm.at[idx_v.at[b, j]],
                    rows_v.at[b, pl.ds(j * 128, 128)],
                    gsem.at[b]).wait()

        def start_write(ch, b):
            pltpu.async_copy(
                rows_v.at[b],
                out_hbm.at[pl.ds((row0 + ch * KSUB) * 128, CHUNK)],
                wsem.at[b])

        def wait_write(ch, b):
            pltpu.make_async_copy(
                rows_v.at[b],
                out_hbm.at[pl.ds((row0 + ch * KSUB) * 128, CHUNK)],
                wsem.at[b]).wait()

        def load_idx(ch, b):
            pltpu.sync_copy(x_hbm.at[pl.ds(row0 + ch * KSUB, KSUB)],
                            idx_v.at[b])

        load_idx(0, 0)
        fire_gathers(0, 0)
        load_idx(1, 1)
        fire_gathers(1, 1)

        def step(t, carry):
            for b in range(2):
                ch = 2 * t + b
                wait_gathers(ch, b)
                start_write(ch, b)
                load_idx(ch + 2, b)
                wait_write(ch, b)
                fire_gathers(ch + 2, b)
            return carry

        lax.fori_loop(0, NCHUNKS // 2 - 1, step, 0)

        for b in range(2):
            ch = NCHUNKS - 2 + b
            wait_gathers(ch, b)
            start_write(ch, b)
        for b in range(2):
            wait_write(NCHUNKS - 2 + b, b)

    return k(x2d, table)


def kernel(x, table):
    x2d = jnp.reshape(x.astype(jnp.int32), (TOTAL // 128, 128))
    out = _sc_gather(x2d, table)
    return jnp.reshape(out, (BATCH, HIST, EMBED_DIM))
